# baseline (speedup 1.0000x reference)
import jax
import jax.numpy as jnp
from jax import lax
from jax.experimental import pallas as pl
from jax.experimental.pallas import tpu as pltpu
import jax.experimental.pallas.tpu_sc as plsc

B, E, A = 4096, 8, 1000
L = 16
NC, NS = 2, 16
NW = NC * NS
BW = B // NW
NCH = BW // L
AB = 8
NBLK = A // AB
NBUF = 3
NB2 = 3
NEG = -3e38


def _sc_body(a_hbm, er_hbm, o_hbm, a_v, e_v, p_v, ms_v, in_sem, out_sem, p2_sem):
    wid = lax.axis_index("s") * NC + lax.axis_index("c")
    bw = pl.multiple_of(wid * BW, BW)
    lane = lax.iota(jnp.int32, L)

    def start_in(j, blk):
        a0 = pl.multiple_of(blk * AB, AB)
        pltpu.async_copy(a_hbm.at[:, pl.ds(a0, AB), pl.ds(bw, BW)], a_v.at[j], in_sem.at[j])
        pltpu.async_copy(er_hbm.at[:, pl.ds(a0, AB), pl.ds(bw, BW)], e_v.at[j], in_sem.at[j])

    def wait_in(j, blk):
        a0 = pl.multiple_of(blk * AB, AB)
        pltpu.make_async_copy(a_hbm.at[:, pl.ds(a0, AB), pl.ds(bw, BW)], a_v.at[j], in_sem.at[j]).wait()
        pltpu.make_async_copy(er_hbm.at[:, pl.ds(a0, AB), pl.ds(bw, BW)], e_v.at[j], in_sem.at[j]).wait()

    def out_slab(j, blk):
        a0 = pl.multiple_of(blk * AB, AB)
        return pltpu.make_async_copy(
            p_v.at[j], o_hbm.at[pl.ds(a0, AB), pl.ds(bw, BW)], out_sem.at[j])

    for j in range(NBUF):
        start_in(j, j)

    def compute_block(j, blk):
        def pa(a, m):
            mo = []
            for c in range(NCH):
                off = c * L
                vals = [a_v[j, e, a, pl.ds(off, L)] for e in range(E)]
                stds = [e_v[j, e, a, pl.ds(off, L)] for e in range(E)]
                while len(vals) > 1:
                    nv, ns_ = [], []
                    for k in range(0, len(vals), 2):
                        take = vals[k + 1] < vals[k]
                        nv.append(jnp.where(take, vals[k + 1], vals[k]))
                        ns_.append(jnp.where(take, stds[k + 1], stds[k]))
                    vals, stds = nv, ns_
                scaled = vals[0] / jnp.maximum(stds[0], 1e-8)
                p_v[j, a, pl.ds(off, L)] = scaled
                mo.append(jnp.maximum(m[c], scaled))
            return tuple(mo)

        m = lax.fori_loop(0, AB, pa,
                          tuple(jnp.full((L,), NEG, jnp.float32) for _ in range(NCH)))

        def pb(a, s):
            so = []
            for c in range(NCH):
                off = c * L
                p = jnp.exp(p_v[j, a, pl.ds(off, L)] - m[c])
                p_v[j, a, pl.ds(off, L)] = p
                so.append(s[c] + p)
            return tuple(so)

        s = lax.fori_loop(0, AB, pb,
                          tuple(jnp.zeros((L,), jnp.float32) for _ in range(NCH)))

        for c in range(NCH):
            st = pl.multiple_of(blk * (2 * BW) + c * L, L)
            ms_v[pl.ds(st, L)] = m[c]
            ms_v[pl.ds(st + BW, L)] = s[c]
        out_slab(j, blk).start()

    NRING = (NBLK - 2) // NBUF

    def blk_body(g, carry):
        for j in range(NBUF):
            blk = g * NBUF + j
            wait_in(j, blk)

            @pl.when(g > 0)
            def _():
                out_slab(j, blk - NBUF).wait()

            compute_block(j, blk)

            @pl.when(g < NRING - 1)
            def _():
                start_in(j, blk + NBUF)
        return carry

    lax.fori_loop(0, NRING, blk_body, 0)
    out_slab(0, NBLK - 5).wait()
    start_in(0, NBLK - 2)
    out_slab(1, NBLK - 4).wait()
    start_in(1, NBLK - 1)
    wait_in(0, NBLK - 2)
    compute_block(0, NBLK - 2)
    wait_in(1, NBLK - 1)
    compute_block(1, NBLK - 1)
    out_slab(2, NBLK - 3).wait()
    out_slab(0, NBLK - 2).wait()
    out_slab(1, NBLK - 1).wait()

    def red(blk, m):
        return tuple(
            jnp.maximum(m[c], ms_v[pl.ds(pl.multiple_of(blk * (2 * BW) + c * L, L), L)])
            for c in range(NCH))

    m_fin = lax.fori_loop(0, NBLK, red,
                          tuple(jnp.full((L,), NEG, jnp.float32) for _ in range(NCH)))

    def tot(blk, s):
        so = []
        for c in range(NCH):
            st = pl.multiple_of(blk * (2 * BW) + c * L, L)
            so.append(s[c] + ms_v[pl.ds(st + BW, L)] * jnp.exp(ms_v[pl.ds(st, L)] - m_fin[c]))
        return tuple(so)

    s_tot = lax.fori_loop(0, NBLK, tot,
                          tuple(jnp.zeros((L,), jnp.float32) for _ in range(NCH)))
    inv = tuple(1.0 / s_tot[c] for c in range(NCH))

    def p2_in(j, blk):
        a0 = pl.multiple_of(blk * AB, AB)
        return pltpu.make_async_copy(
            o_hbm.at[pl.ds(a0, AB), pl.ds(bw, BW)], p_v.at[j], p2_sem.at[j])

    def rescale_block(j, blk):
        f = []
        for c in range(NCH):
            st = pl.multiple_of(blk * (2 * BW) + c * L, L)
            f.append(jnp.exp(ms_v[pl.ds(st, L)] - m_fin[c]) * inv[c])

        def pc(a, carry2):
            for c in range(NCH):
                off = c * L
                p_v[j, a, pl.ds(off, L)] = p_v[j, a, pl.ds(off, L)] * f[c]
            return carry2

        lax.fori_loop(0, AB, pc, 0)
        out_slab(j, blk).start()

    for j in range(2):
        p2_in(j, j).start()
    NRING2 = (NBLK - 2) // NB2

    def blk2_body(g, carry):
        for j in range(NB2):
            blk = g * NB2 + j
            p2_in(j, blk).wait()
            rescale_block(j, blk)
            jn = (j + 2) % NB2

            if j == 0:
                @pl.when(g > 0)
                def _():
                    out_slab((j + 2) % NB2, blk - 1).wait()
                    p2_in(jn, blk + 2).start()

                @pl.when(g == 0)
                def _():
                    p2_in(jn, blk + 2).start()
            else:
                out_slab(j - 1, blk - 1).wait()
                p2_in(jn, blk + 2).start()
        return carry

    lax.fori_loop(0, NRING2, blk2_body, 0)
    p2_in(0, NBLK - 2).wait()
    rescale_block(0, NBLK - 2)
    p2_in(1, NBLK - 1).wait()
    rescale_block(1, NBLK - 1)
    out_slab(2, NBLK - 3).wait()
    out_slab(0, NBLK - 2).wait()
    out_slab(1, NBLK - 1).wait()


@jax.jit
def _sc_call(at, et):
    return pl.kernel(
        _sc_body,
        out_type=jax.ShapeDtypeStruct((A, B), jnp.float32),
        mesh=plsc.VectorSubcoreMesh(
            core_axis_name="c", subcore_axis_name="s",
            num_cores=NC, num_subcores=NS,
        ),
        scratch_types=[
            pltpu.VMEM((NBUF, E, AB, BW), jnp.float32),
            pltpu.VMEM((NBUF, E, AB, BW), jnp.float32),
            pltpu.VMEM((NB2, AB, BW), jnp.float32),
            pltpu.VMEM((NBLK * 2 * BW,), jnp.float32),
            pltpu.SemaphoreType.DMA((NBUF,)),
            pltpu.SemaphoreType.DMA((NB2,)),
            pltpu.SemaphoreType.DMA((NB2,)),
        ],
        compiler_params=pltpu.CompilerParams(
            use_tc_tiling_on_sc=True, needs_layout_passes=False,
        ),
    )(at, et)


def kernel(action, explore_rate, step, obs):
    del step, obs
    at = jnp.transpose(action, (1, 2, 0))
    et = jnp.transpose(explore_rate, (1, 2, 0))
    out_t = _sc_call(at, et)
    return jnp.transpose(out_t, (1, 0))


# speedup vs baseline: 1.0581x; 1.0581x over previous
import jax
import jax.numpy as jnp
from jax import lax
from jax.experimental import pallas as pl
from jax.experimental.pallas import tpu as pltpu
import jax.experimental.pallas.tpu_sc as plsc

B, E, A = 4096, 8, 1000
L = 16
NC, NS = 2, 16
NW = NC * NS
BW = B // NW
NCH = BW // L
AB = 8
NBLK = A // AB
NBUF = 2
NB2 = 3
NEG = -3e38


def _sc_body(a_hbm, er_hbm, o_hbm, a_v, e_v, p_v, ms_v, in_sem, out_sem, p2_sem):
    wid = lax.axis_index("s") * NC + lax.axis_index("c")
    bw = pl.multiple_of(wid * BW, BW)
    lane = lax.iota(jnp.int32, L)

    def start_in(j, blk):
        a0 = pl.multiple_of(blk * AB, AB)
        pltpu.async_copy(a_hbm.at[:, pl.ds(a0, AB), pl.ds(bw, BW)], a_v.at[j], in_sem.at[j])
        pltpu.async_copy(er_hbm.at[:, pl.ds(a0, AB), pl.ds(bw, BW)], e_v.at[j], in_sem.at[j])

    def wait_in(j, blk):
        a0 = pl.multiple_of(blk * AB, AB)
        pltpu.make_async_copy(a_hbm.at[:, pl.ds(a0, AB), pl.ds(bw, BW)], a_v.at[j], in_sem.at[j]).wait()
        pltpu.make_async_copy(er_hbm.at[:, pl.ds(a0, AB), pl.ds(bw, BW)], e_v.at[j], in_sem.at[j]).wait()

    def out_slab(j, blk):
        a0 = pl.multiple_of(blk * AB, AB)
        return pltpu.make_async_copy(
            p_v.at[j], o_hbm.at[pl.ds(a0, AB), pl.ds(bw, BW)], out_sem.at[j])

    for j in range(NBUF):
        start_in(j, j)

    def compute_block(j, blk):
        def pa(a, m):
            mo = []
            for c in range(NCH):
                off = c * L
                vals = [a_v[j, e, a, pl.ds(off, L)] for e in range(E)]
                stds = [e_v[j, e, a, pl.ds(off, L)] for e in range(E)]
                while len(vals) > 1:
                    nv, ns_ = [], []
                    for k in range(0, len(vals), 2):
                        take = vals[k + 1] < vals[k]
                        nv.append(jnp.where(take, vals[k + 1], vals[k]))
                        ns_.append(jnp.where(take, stds[k + 1], stds[k]))
                    vals, stds = nv, ns_
                scaled = vals[0] / jnp.maximum(stds[0], 1e-8)
                p_v[j, a, pl.ds(off, L)] = scaled
                mo.append(jnp.maximum(m[c], scaled))
            return tuple(mo)

        m = lax.fori_loop(0, AB, pa,
                          tuple(jnp.full((L,), NEG, jnp.float32) for _ in range(NCH)))

        def pb(a, s):
            so = []
            for c in range(NCH):
                off = c * L
                p = jnp.exp(p_v[j, a, pl.ds(off, L)] - m[c])
                p_v[j, a, pl.ds(off, L)] = p
                so.append(s[c] + p)
            return tuple(so)

        s = lax.fori_loop(0, AB, pb,
                          tuple(jnp.zeros((L,), jnp.float32) for _ in range(NCH)))

        for c in range(NCH):
            st = pl.multiple_of(blk * (2 * BW) + c * L, L)
            ms_v[pl.ds(st, L)] = m[c]
            ms_v[pl.ds(st + BW, L)] = s[c]
        out_slab(j, blk).start()

    NRING = NBLK // NBUF

    def blk_body(g, carry):
        for j in range(NBUF):
            blk = g * NBUF + j
            wait_in(j, blk)

            @pl.when(g > 0)
            def _():
                out_slab(j, blk - NBUF).wait()

            compute_block(j, blk)

            @pl.when(g < NRING - 1)
            def _():
                start_in(j, blk + NBUF)
        return carry

    lax.fori_loop(0, NRING, blk_body, 0)
    start_in(0, NBLK - 1)
    wait_in(0, NBLK - 1)
    out_slab(0, NBLK - 3).wait()
    compute_block(0, NBLK - 1)
    out_slab(1, NBLK - 2).wait()
    out_slab(0, NBLK - 1).wait()

    def red(blk, m):
        return tuple(
            jnp.maximum(m[c], ms_v[pl.ds(pl.multiple_of(blk * (2 * BW) + c * L, L), L)])
            for c in range(NCH))

    m_fin = lax.fori_loop(0, NBLK, red,
                          tuple(jnp.full((L,), NEG, jnp.float32) for _ in range(NCH)))

    def tot(blk, s):
        so = []
        for c in range(NCH):
            st = pl.multiple_of(blk * (2 * BW) + c * L, L)
            so.append(s[c] + ms_v[pl.ds(st + BW, L)] * jnp.exp(ms_v[pl.ds(st, L)] - m_fin[c]))
        return tuple(so)

    s_tot = lax.fori_loop(0, NBLK, tot,
                          tuple(jnp.zeros((L,), jnp.float32) for _ in range(NCH)))
    inv = tuple(1.0 / s_tot[c] for c in range(NCH))

    def p2_in(j, blk):
        a0 = pl.multiple_of(blk * AB, AB)
        return pltpu.make_async_copy(
            o_hbm.at[pl.ds(a0, AB), pl.ds(bw, BW)], p_v.at[j], p2_sem.at[j])

    def rescale_block(j, blk):
        f = []
        for c in range(NCH):
            st = pl.multiple_of(blk * (2 * BW) + c * L, L)
            f.append(jnp.exp(ms_v[pl.ds(st, L)] - m_fin[c]) * inv[c])

        def pc(a, carry2):
            for c in range(NCH):
                off = c * L
                p_v[j, a, pl.ds(off, L)] = p_v[j, a, pl.ds(off, L)] * f[c]
            return carry2

        lax.fori_loop(0, AB, pc, 0)
        out_slab(j, blk).start()

    for j in range(2):
        p2_in(j, j).start()
    NRING2 = (NBLK - 2) // NB2

    def blk2_body(g, carry):
        for j in range(NB2):
            blk = g * NB2 + j
            p2_in(j, blk).wait()
            rescale_block(j, blk)
            jn = (j + 2) % NB2

            if j == 0:
                @pl.when(g > 0)
                def _():
                    out_slab((j + 2) % NB2, blk - 1).wait()
                    p2_in(jn, blk + 2).start()

                @pl.when(g == 0)
                def _():
                    p2_in(jn, blk + 2).start()
            else:
                out_slab(j - 1, blk - 1).wait()
                p2_in(jn, blk + 2).start()
        return carry

    lax.fori_loop(0, NRING2, blk2_body, 0)
    p2_in(0, NBLK - 2).wait()
    rescale_block(0, NBLK - 2)
    p2_in(1, NBLK - 1).wait()
    rescale_block(1, NBLK - 1)
    out_slab(2, NBLK - 3).wait()
    out_slab(0, NBLK - 2).wait()
    out_slab(1, NBLK - 1).wait()


@jax.jit
def _sc_call(at, et):
    return pl.kernel(
        _sc_body,
        out_type=jax.ShapeDtypeStruct((A, B), jnp.float32),
        mesh=plsc.VectorSubcoreMesh(
            core_axis_name="c", subcore_axis_name="s",
            num_cores=NC, num_subcores=NS,
        ),
        scratch_types=[
            pltpu.VMEM((NBUF, E, AB, BW), jnp.float32),
            pltpu.VMEM((NBUF, E, AB, BW), jnp.float32),
            pltpu.VMEM((NB2, AB, BW), jnp.float32),
            pltpu.VMEM((NBLK * 2 * BW,), jnp.float32),
            pltpu.SemaphoreType.DMA((NBUF,)),
            pltpu.SemaphoreType.DMA((NB2,)),
            pltpu.SemaphoreType.DMA((NB2,)),
        ],
        compiler_params=pltpu.CompilerParams(
            use_tc_tiling_on_sc=True, needs_layout_passes=False,
        ),
    )(at, et)


def kernel(action, explore_rate, step, obs):
    del step, obs
    at = jnp.transpose(action, (1, 2, 0))
    et = jnp.transpose(explore_rate, (1, 2, 0))
    out_t = _sc_call(at, et)
    return jnp.transpose(out_t, (1, 0))
